# baseline (speedup 1.0000x reference)
import functools

import jax
import jax.numpy as jnp
from jax import lax
from jax.experimental import pallas as pl
from jax.experimental.pallas import tpu as pltpu
from jax.experimental.pallas import tpu_sc as plsc

B, P, V = 128, 16, 1024
N = P * P
K = 32
NEG_INF = float("-inf")


def _dense_body(logits_ref, mask_ref, u_sample_ref, u_conf_ref,
                conf_ref, pred_ref):
    x = logits_ref[...]
    u = u_sample_ref[...]
    maskv = mask_ref[0]
    uc = u_conf_ref[0]

    m = jnp.max(x, axis=-1, keepdims=True)
    e = jnp.exp(x - m)
    s = jnp.sum(e, axis=-1, keepdims=True)

    den = -jnp.log(u + 1e-9) + 1e-9
    r = (e + 1e-12 * s) / den
    pred = jnp.argmax(r, axis=-1).astype(jnp.int32)

    col = jax.lax.broadcasted_iota(jnp.int32, x.shape, 1)
    e_pred = jnp.sum(jnp.where(col == pred[:, None], e, 0.0), axis=-1)
    conf_p = e_pred / s[:, 0]

    gc = -jnp.log(-jnp.log(uc + 1e-9) + 1e-9)
    conf = jnp.log(conf_p + 1e-12)[None, :] + 4.5 * gc
    conf = jnp.where(maskv != 0.0, conf, NEG_INF)

    conf_ref[0] = conf
    pred_ref[0] = pred[None, :]


_SC_INFO = plsc.get_sparse_core_info()
NW = _SC_INFO.num_cores * _SC_INFO.num_subcores
NB = N // 16


def _make_select_sc(rows):
    rpw = rows // NW

    def body(conf_hbm, mask_hbm, pred_hbm, code_hbm,
             code_out, mask_out, tresh_out,
             conf_v, mask_v, pred_v, code_v,
             pad_v, nmask_v, ncode_v, tresh_v, sem):
        wid = lax.axis_index("s") * _SC_INFO.num_cores + lax.axis_index("c")
        base = wid * rpw
        copies = [
            pltpu.async_copy(conf_hbm.at[pl.ds(base, rpw)], conf_v, sem),
            pltpu.async_copy(mask_hbm.at[pl.ds(base, rpw)], mask_v, sem),
            pltpu.async_copy(pred_hbm.at[pl.ds(base, rpw)], pred_v, sem),
            pltpu.async_copy(code_hbm.at[pl.ds(base, rpw)], code_v, sem),
        ]
        for c in copies:
            c.wait()

        lanes = lax.iota(jnp.int32, 16)
        neg = jnp.full((16,), NEG_INF, jnp.float32)
        for strip in (0, 48):
            pad_v[pl.ds(strip, 16)] = neg
            pad_v[pl.ds(strip + 32, 16)] = neg

        def splat_max(v, strip):
            v = jnp.maximum(v, lax.rev(v, (0,)))
            for sft in (1, 2, 4):
                pad_v[pl.ds(strip + 16, 16)] = v
                lo = pad_v[pl.ds(strip + 16 - sft, 16)]
                hi = pad_v[pl.ds(strip + 16 + sft, 16)]
                v = jnp.maximum(jnp.maximum(v, lo), hi)
            return v

        for rp in range(rpw // 2):
            ra, rb = 2 * rp, 2 * rp + 1
            wa = [conf_v[ra, pl.ds(16 * j, 16)] for j in range(NB)]
            wb = [conf_v[rb, pl.ds(16 * j, 16)] for j in range(NB)]
            zf = jnp.zeros((16,), jnp.float32)

            def step(t, carry):
                wa, wb, ta0, ta1, tb0, tb1 = carry
                ma = wa[0]
                mb = wb[0]
                for j in range(1, NB):
                    ma = jnp.maximum(ma, wa[j])
                    mb = jnp.maximum(mb, wb[j])
                ga = splat_max(ma, 0)
                gb = splat_max(mb, 48)
                wa = [jnp.where(w == ga, NEG_INF, w) for w in wa]
                wb = [jnp.where(w == gb, NEG_INF, w) for w in wb]
                ta0 = jnp.where(lanes == t, ga, ta0)
                ta1 = jnp.where(lanes == (t - 16), ga, ta1)
                tb0 = jnp.where(lanes == t, gb, tb0)
                tb1 = jnp.where(lanes == (t - 16), gb, tb1)
                return wa, wb, ta0, ta1, tb0, tb1

            wa, wb, ta0, ta1, tb0, tb1 = lax.fori_loop(
                0, K, step, (wa, wb, zf, zf, zf, zf))

            for rr, work, tv0, tv1 in ((ra, wa, ta0, ta1),
                                       (rb, wb, tb0, tb1)):
                tresh = splat_max(jnp.where(lanes == 15, tv1, NEG_INF), 0)
                for j in range(NB):
                    ds = pl.ds(16 * j, 16)
                    c = conf_v[rr, ds]
                    mk = mask_v[rr, ds]
                    sel = (c >= tresh) & (mk != 0.0)
                    ncode_v[rr, ds] = jnp.where(sel, pred_v[rr, ds],
                                                code_v[rr, ds])
                    nmask_v[rr, ds] = jnp.where(work[j] == NEG_INF, 0.0, mk)
                tresh_v[rr, pl.ds(0, 16)] = tv0
                tresh_v[rr, pl.ds(16, 16)] = tv1

        pltpu.sync_copy(ncode_v, code_out.at[pl.ds(base, rpw)])
        pltpu.sync_copy(nmask_v, mask_out.at[pl.ds(base, rpw)])
        pltpu.sync_copy(tresh_v, tresh_out.at[pl.ds(base, rpw)])

    return functools.partial(
        pl.kernel,
        mesh=plsc.VectorSubcoreMesh(core_axis_name="c", subcore_axis_name="s"),
        out_type=(
            jax.ShapeDtypeStruct((rows, N), jnp.int32),
            jax.ShapeDtypeStruct((rows, N), jnp.float32),
            jax.ShapeDtypeStruct((rows, K), jnp.float32),
        ),
        scratch_types=[
            pltpu.VMEM((rpw, N), jnp.float32),
            pltpu.VMEM((rpw, N), jnp.float32),
            pltpu.VMEM((rpw, N), jnp.int32),
            pltpu.VMEM((rpw, N), jnp.int32),
            pltpu.VMEM((96,), jnp.float32),
            pltpu.VMEM((rpw, N), jnp.float32),
            pltpu.VMEM((rpw, N), jnp.int32),
            pltpu.VMEM((rpw, K), jnp.float32),
            pltpu.SemaphoreType.DMA,
        ],
    )(body)


G = 32
RB = (B * N) // G
HB = B // 2
GH = G // 2

_select_sc_half = _make_select_sc(HB)


def kernel(logits, mask, u_sample, u_conf, code, k):
    del k
    logits2 = logits.reshape(B * N, V)
    u_sample2 = u_sample.reshape(B * N, V)
    mask3 = mask.reshape(G, 1, RB)
    u_conf3 = u_conf.reshape(G, 1, RB)
    code2 = code.reshape(B, N)

    outs = []
    for h in range(2):
        conf, pred = pl.pallas_call(
            _dense_body,
            grid=(GH,),
            in_specs=[
                pl.BlockSpec((RB, V), lambda b, _h=h: (b + _h * GH, 0)),
                pl.BlockSpec((1, 1, RB), lambda b, _h=h: (b + _h * GH, 0, 0)),
                pl.BlockSpec((RB, V), lambda b, _h=h: (b + _h * GH, 0)),
                pl.BlockSpec((1, 1, RB), lambda b, _h=h: (b + _h * GH, 0, 0)),
            ],
            out_specs=(
                pl.BlockSpec((1, 1, RB), lambda b: (b, 0, 0)),
                pl.BlockSpec((1, 1, RB), lambda b: (b, 0, 0)),
            ),
            out_shape=(
                jax.ShapeDtypeStruct((GH, 1, RB), jnp.float32),
                jax.ShapeDtypeStruct((GH, 1, RB), jnp.int32),
            ),
            compiler_params=pltpu.CompilerParams(
                dimension_semantics=("parallel",)),
        )(logits2, mask3, u_sample2, u_conf3)

        lo = h * HB
        outs.append(_select_sc_half(
            conf.reshape(HB, N), mask[lo:lo + HB],
            pred.reshape(HB, N), code2[lo:lo + HB]))

    new_code = jnp.concatenate([outs[0][0], outs[1][0]], axis=0)
    new_mask = jnp.concatenate([outs[0][1], outs[1][1]], axis=0)
    tresh_conf = jnp.concatenate([outs[0][2], outs[1][2]], axis=0)
    return (new_code.reshape(B, P, P), new_mask, tresh_conf)


# speedup vs baseline: 1.0535x; 1.0535x over previous
import functools

import jax
import jax.numpy as jnp
from jax import lax
from jax.experimental import pallas as pl
from jax.experimental.pallas import tpu as pltpu
from jax.experimental.pallas import tpu_sc as plsc

B, P, V = 128, 16, 1024
N = P * P
K = 32
NEG_INF = float("-inf")


def _dense_body(logits_ref, mask_ref, u_sample_ref, u_conf_ref,
                conf_ref, pred_ref):
    x = logits_ref[...]
    u = u_sample_ref[...]
    maskv = mask_ref[0]
    uc = u_conf_ref[0]

    m = jnp.max(x, axis=-1, keepdims=True)
    e = jnp.exp(x - m)
    s = jnp.sum(e, axis=-1, keepdims=True)

    den = -jnp.log(u + 1e-9) + 1e-9
    r = (e + 1e-12 * s) / den
    pred = jnp.argmax(r, axis=-1).astype(jnp.int32)

    col = jax.lax.broadcasted_iota(jnp.int32, x.shape, 1)
    e_pred = jnp.sum(jnp.where(col == pred[:, None], e, 0.0), axis=-1)
    conf_p = e_pred / s[:, 0]

    gc = -jnp.log(-jnp.log(uc + 1e-9) + 1e-9)
    conf = jnp.log(conf_p + 1e-12)[None, :] + 4.5 * gc
    conf = jnp.where(maskv != 0.0, conf, NEG_INF)

    conf_ref[0] = conf
    pred_ref[0] = pred[None, :]


_SC_INFO = plsc.get_sparse_core_info()
NW = _SC_INFO.num_cores * _SC_INFO.num_subcores
RPW = B // NW
NB = N // 16


def _select_sc_body(conf_hbm, mask_hbm, pred_hbm, code_hbm,
                    code_out, mask_out, tresh_out,
                    conf_v, mask_v, pred_v, code_v,
                    pad_v, nmask_v, ncode_v, tresh_v, sem):
    wid = lax.axis_index("s") * _SC_INFO.num_cores + lax.axis_index("c")
    base = wid * RPW
    copies = [
        pltpu.async_copy(conf_hbm.at[pl.ds(base, RPW)], conf_v, sem),
        pltpu.async_copy(mask_hbm.at[pl.ds(base, RPW)], mask_v, sem),
        pltpu.async_copy(pred_hbm.at[pl.ds(base, RPW)], pred_v, sem),
        pltpu.async_copy(code_hbm.at[pl.ds(base, RPW)], code_v, sem),
    ]
    for c in copies:
        c.wait()

    lanes = lax.iota(jnp.int32, 16)
    neg = jnp.full((16,), NEG_INF, jnp.float32)
    for strip in (0, 48):
        pad_v[pl.ds(strip, 16)] = neg
        pad_v[pl.ds(strip + 32, 16)] = neg

    def splat_max(v, strip):
        v = jnp.maximum(v, lax.rev(v, (0,)))
        for sft in (1, 2, 4):
            pad_v[pl.ds(strip + 16, 16)] = v
            lo = pad_v[pl.ds(strip + 16 - sft, 16)]
            hi = pad_v[pl.ds(strip + 16 + sft, 16)]
            v = jnp.maximum(jnp.maximum(v, lo), hi)
        return v

    for rp in range(RPW // 2):
        ra, rb = 2 * rp, 2 * rp + 1
        wa = [conf_v[ra, pl.ds(16 * j, 16)] for j in range(NB)]
        wb = [conf_v[rb, pl.ds(16 * j, 16)] for j in range(NB)]
        zf = jnp.zeros((16,), jnp.float32)

        def step(t, carry):
            wa, wb, ta0, ta1, tb0, tb1 = carry
            ma = wa[0]
            mb = wb[0]
            for j in range(1, NB):
                ma = jnp.maximum(ma, wa[j])
                mb = jnp.maximum(mb, wb[j])
            ga = splat_max(ma, 0)
            gb = splat_max(mb, 48)
            wa = [jnp.where(w == ga, NEG_INF, w) for w in wa]
            wb = [jnp.where(w == gb, NEG_INF, w) for w in wb]
            ta0 = jnp.where(lanes == t, ga, ta0)
            ta1 = jnp.where(lanes == (t - 16), ga, ta1)
            tb0 = jnp.where(lanes == t, gb, tb0)
            tb1 = jnp.where(lanes == (t - 16), gb, tb1)
            return wa, wb, ta0, ta1, tb0, tb1

        wa, wb, ta0, ta1, tb0, tb1 = lax.fori_loop(
            0, K, step, (wa, wb, zf, zf, zf, zf))

        for rr, work, tv0, tv1 in ((ra, wa, ta0, ta1), (rb, wb, tb0, tb1)):
            tresh = splat_max(jnp.where(lanes == 15, tv1, NEG_INF), 0)
            for j in range(NB):
                ds = pl.ds(16 * j, 16)
                c = conf_v[rr, ds]
                mk = mask_v[rr, ds]
                sel = (c >= tresh) & (mk != 0.0)
                ncode_v[rr, ds] = jnp.where(sel, pred_v[rr, ds],
                                            code_v[rr, ds])
                nmask_v[rr, ds] = jnp.where(work[j] == NEG_INF, 0.0, mk)
            tresh_v[rr, pl.ds(0, 16)] = tv0
            tresh_v[rr, pl.ds(16, 16)] = tv1

    pltpu.sync_copy(ncode_v, code_out.at[pl.ds(base, RPW)])
    pltpu.sync_copy(nmask_v, mask_out.at[pl.ds(base, RPW)])
    pltpu.sync_copy(tresh_v, tresh_out.at[pl.ds(base, RPW)])


_select_sc = functools.partial(
    pl.kernel,
    mesh=plsc.VectorSubcoreMesh(core_axis_name="c", subcore_axis_name="s"),
    out_type=(
        jax.ShapeDtypeStruct((B, N), jnp.int32),
        jax.ShapeDtypeStruct((B, N), jnp.float32),
        jax.ShapeDtypeStruct((B, K), jnp.float32),
    ),
    scratch_types=[
        pltpu.VMEM((RPW, N), jnp.float32),
        pltpu.VMEM((RPW, N), jnp.float32),
        pltpu.VMEM((RPW, N), jnp.int32),
        pltpu.VMEM((RPW, N), jnp.int32),
        pltpu.VMEM((96,), jnp.float32),
        pltpu.VMEM((RPW, N), jnp.float32),
        pltpu.VMEM((RPW, N), jnp.int32),
        pltpu.VMEM((RPW, K), jnp.float32),
        pltpu.SemaphoreType.DMA,
    ],
)(_select_sc_body)


G = 32
RB = (B * N) // G


def kernel(logits, mask, u_sample, u_conf, code, k):
    del k
    logits2 = logits.reshape(B * N, V)
    u_sample2 = u_sample.reshape(B * N, V)
    mask3 = mask.reshape(G, 1, RB)
    u_conf3 = u_conf.reshape(G, 1, RB)

    conf, pred = pl.pallas_call(
        _dense_body,
        grid=(G,),
        in_specs=[
            pl.BlockSpec((RB, V), lambda b: (b, 0)),
            pl.BlockSpec((1, 1, RB), lambda b: (b, 0, 0)),
            pl.BlockSpec((RB, V), lambda b: (b, 0)),
            pl.BlockSpec((1, 1, RB), lambda b: (b, 0, 0)),
        ],
        out_specs=(
            pl.BlockSpec((1, 1, RB), lambda b: (b, 0, 0)),
            pl.BlockSpec((1, 1, RB), lambda b: (b, 0, 0)),
        ),
        out_shape=(
            jax.ShapeDtypeStruct((G, 1, RB), jnp.float32),
            jax.ShapeDtypeStruct((G, 1, RB), jnp.int32),
        ),
        compiler_params=pltpu.CompilerParams(
            dimension_semantics=("parallel",)),
    )(logits2, mask3, u_sample2, u_conf3)

    new_code, new_mask, tresh_conf = _select_sc(
        conf.reshape(B, N), mask, pred.reshape(B, N), code.reshape(B, N))

    return (new_code.reshape(B, P, P), new_mask, tresh_conf)
